# Initial kernel scaffold; baseline (speedup 1.0000x reference)
import jax
import jax.numpy as jnp
from jax.experimental import pallas as pl


def kernel(x):
    raise NotImplementedError("write your pallas kernel here")


# speedup vs baseline: 375.2206x; 375.2206x over previous
import jax
import jax.numpy as jnp
from jax.experimental import pallas as pl

FRAME = 2048
HOP = 512
FV = 4096
BV = 4
PAD = 768
H = 512
NC = FV // H
NHB = FV // 8


def _body(main_ref, lo_ref, hi_ref, out_ref):
    c = pl.program_id(1)
    nc = pl.num_programs(1)
    main = main_ref[0]
    lo = lo_ref[0, 6:8, :]
    hi = hi_ref[0, 0:2, :]
    lo = jnp.where(c == 0, 0.0, lo)
    hi = jnp.where(c == nc - 1, 0.0, hi)
    xw = jnp.concatenate([lo, main, hi], axis=0)
    p = (xw[3:H + 4, 0:512] + xw[2:H + 3, 512:1024]
         + xw[1:H + 2, 1024:1536] + xw[0:H + 1, 1536:2048])
    hof = (jax.lax.broadcasted_iota(jnp.int32, (H + 1, 1), 0) + (c * H + 1)).astype(jnp.float32)
    cnt = jnp.minimum(jnp.minimum(hof + 1.0, 4.0), 4099.0 - hof)
    pn = p / cnt
    out_ref[0] = jnp.concatenate([pn[0:H, 256:512], pn[1:H + 1, 0:256]], axis=1)


def kernel(x):
    hb = H // 8
    out3 = pl.pallas_call(
        _body,
        grid=(BV, NC),
        in_specs=[
            pl.BlockSpec((1, H, FRAME), lambda b, c: (b, c, 0)),
            pl.BlockSpec((1, 8, FRAME), lambda b, c: (b, jnp.maximum(c * hb - 1, 0), 0)),
            pl.BlockSpec((1, 8, FRAME), lambda b, c: (b, jnp.minimum(c * hb + hb, NHB - 1), 0)),
        ],
        out_specs=pl.BlockSpec((1, H, HOP), lambda b, c: (b, c, 0)),
        out_shape=jax.ShapeDtypeStruct((BV, FV, HOP), x.dtype),
    )(x, x, x)
    return out3.reshape(BV, FV * HOP)
